# baseline (speedup 1.0000x reference)
import functools

import jax
import jax.numpy as jnp
from jax import lax
from jax.experimental import pallas as pl
from jax.experimental.pallas import tpu as pltpu
from jax.experimental.pallas import tpu_sc as plsc

NUM_SIGNALS = 100000
ROW_WIDTH = 70
BATCH = 4096

_info = plsc.get_sparse_core_info()
_NC, _NS = _info.num_cores, _info.num_subcores
_NW = _NC * _NS
_UNROLL = 4
_NGRP = BATCH // (16 * _UNROLL)


def _make_gather():
    mesh = plsc.VectorSubcoreMesh(core_axis_name="c", subcore_axis_name="s")

    @functools.partial(
        pl.kernel,
        mesh=mesh,
        out_type=jax.ShapeDtypeStruct((ROW_WIDTH, BATCH), jnp.float32),
        scratch_types=[
            pltpu.VMEM((BATCH,), jnp.int32),
            pltpu.VMEM((NUM_SIGNALS,), jnp.float32),
            pltpu.VMEM((BATCH,), jnp.float32),
            pltpu.VMEM((160,), jnp.float32),
            pltpu.SemaphoreType.DMA,
        ],
        compiler_params=pltpu.CompilerParams(
            use_tc_tiling_on_sc=True,
            needs_layout_passes=False,
            skip_device_barrier=True,
        ),
    )
    def gather_kernel(idx_hbm, table_hbm, out_hbm, idx_v, plane_v, res_v,
                      tail_v, sem):
        wid = lax.axis_index("s") * _NC + lax.axis_index("c")
        pltpu.sync_copy(idx_hbm, idx_v)
        _QS = [(0, 33280), (33280, 33280), (66560, 33280)]

        def do_plane(j):
            for off, sz in _QS:
                pltpu.make_async_copy(
                    table_hbm.at[j, pl.ds(off, sz)],
                    plane_v.at[pl.ds(off, sz)],
                    sem,
                ).start()
            pltpu.make_async_copy(
                table_hbm.at[j, pl.ds(99840, 160)], tail_v, sem
            ).start()
            for off, sz in _QS:
                pltpu.make_async_copy(
                    table_hbm.at[0, pl.ds(off, sz)],
                    plane_v.at[pl.ds(off, sz)],
                    sem,
                ).wait()
            pltpu.make_async_copy(
                table_hbm.at[0, pl.ds(99840, 160)], tail_v, sem
            ).wait()
            for k in range(10):
                plane_v[pl.ds(99840 + k * 16, 16)] = tail_v[pl.ds(k * 16, 16)]

            def gather_grp(k, carry):
                for u in range(_UNROLL):
                    o = (k * _UNROLL + u) * 16
                    g = plsc.load_gather(plane_v, [idx_v[pl.ds(o, 16)]])
                    res_v[pl.ds(o, 16)] = g
                return carry

            lax.fori_loop(0, _NGRP, gather_grp, 0)
            pltpu.sync_copy(res_v, out_hbm.at[j])

        do_plane(wid)
        do_plane(wid + _NW)

        @pl.when(wid < ROW_WIDTH - 2 * _NW)
        def _():
            do_plane(wid + 2 * _NW)

    return gather_kernel


_gather = _make_gather()


def kernel(idx, appearance):
    out_t = _gather(idx.astype(jnp.int32), appearance.T)
    return out_t.T


# speedup vs baseline: 1.0083x; 1.0083x over previous
import functools

import jax
import jax.numpy as jnp
from jax import lax
from jax.experimental import pallas as pl
from jax.experimental.pallas import tpu as pltpu
from jax.experimental.pallas import tpu_sc as plsc

NUM_SIGNALS = 100000
ROW_WIDTH = 70
BATCH = 4096

_info = plsc.get_sparse_core_info()
_NC, _NS = _info.num_cores, _info.num_subcores
_NW = _NC * _NS
_UNROLL = 8
_NGRP = BATCH // (16 * _UNROLL)


def _make_gather():
    mesh = plsc.VectorSubcoreMesh(core_axis_name="c", subcore_axis_name="s")

    @functools.partial(
        pl.kernel,
        mesh=mesh,
        out_type=jax.ShapeDtypeStruct((ROW_WIDTH, BATCH), jnp.float32),
        scratch_types=[
            pltpu.VMEM((BATCH,), jnp.int32),
            pltpu.VMEM((NUM_SIGNALS,), jnp.float32),
            pltpu.VMEM((BATCH,), jnp.float32),
        ],
        compiler_params=pltpu.CompilerParams(
            use_tc_tiling_on_sc=True,
            needs_layout_passes=False,
            skip_device_barrier=True,
        ),
    )
    def gather_kernel(idx_hbm, table_hbm, out_hbm, idx_v, plane_v, res_v):
        wid = lax.axis_index("s") * _NC + lax.axis_index("c")
        pltpu.sync_copy(idx_hbm, idx_v)

        def do_plane(j):
            pltpu.sync_copy(table_hbm.at[j], plane_v)

            def gather_grp(k, carry):
                for u in range(_UNROLL):
                    o = (k * _UNROLL + u) * 16
                    g = plsc.load_gather(plane_v, [idx_v[pl.ds(o, 16)]])
                    res_v[pl.ds(o, 16)] = g
                return carry

            lax.fori_loop(0, _NGRP, gather_grp, 0)
            pltpu.sync_copy(res_v, out_hbm.at[j])

        do_plane(wid)
        do_plane(wid + _NW)

        @pl.when(wid < ROW_WIDTH - 2 * _NW)
        def _():
            do_plane(wid + 2 * _NW)

    return gather_kernel


_gather = _make_gather()


def kernel(idx, appearance):
    out_t = _gather(idx.astype(jnp.int32), appearance.T)
    return out_t.T


# speedup vs baseline: 1.0693x; 1.0605x over previous
import functools

import jax
import jax.numpy as jnp
from jax import lax
from jax.experimental import pallas as pl
from jax.experimental.pallas import tpu as pltpu
from jax.experimental.pallas import tpu_sc as plsc

NUM_SIGNALS = 100000
ROW_WIDTH = 70
BATCH = 4096

_info = plsc.get_sparse_core_info()
_NC, _NS = _info.num_cores, _info.num_subcores
_NW = _NC * _NS
_UNROLL = 8
_NGRP = BATCH // (16 * _UNROLL)


def _make_gather():
    mesh = plsc.VectorSubcoreMesh(core_axis_name="c", subcore_axis_name="s")

    @functools.partial(
        pl.kernel,
        mesh=mesh,
        out_type=jax.ShapeDtypeStruct((ROW_WIDTH, BATCH), jnp.float32),
        scratch_types=[
            pltpu.VMEM((BATCH,), jnp.int32),
            pltpu.VMEM((NUM_SIGNALS,), jnp.float32),
            pltpu.VMEM((BATCH,), jnp.float32),
        ],
        compiler_params=pltpu.CompilerParams(
            use_tc_tiling_on_sc=True,
            needs_layout_passes=False,
            skip_device_barrier=True,
        ),
    )
    def gather_kernel(idx_hbm, table_hbm, out_hbm, idx_v, plane_v, res_v):
        wid = lax.axis_index("s") * _NC + lax.axis_index("c")
        pltpu.sync_copy(idx_hbm, idx_v)

        def do_plane(j):
            pltpu.sync_copy(table_hbm.at[j], plane_v)

            @plsc.parallel_loop(0, BATCH, step=16, unroll=_UNROLL)
            def _(o):
                g = plsc.load_gather(plane_v, [idx_v[pl.ds(o, 16)]])
                res_v[pl.ds(o, 16)] = g

            pltpu.sync_copy(res_v, out_hbm.at[j])

        do_plane(wid)
        do_plane(wid + _NW)

        @pl.when(wid < ROW_WIDTH - 2 * _NW)
        def _():
            do_plane(wid + 2 * _NW)

    return gather_kernel


_gather = _make_gather()


def kernel(idx, appearance):
    out_t = _gather(idx.astype(jnp.int32), appearance.T)
    return out_t.T


# speedup vs baseline: 1.0872x; 1.0168x over previous
import functools

import jax
import jax.numpy as jnp
from jax import lax
from jax.experimental import pallas as pl
from jax.experimental.pallas import tpu as pltpu
from jax.experimental.pallas import tpu_sc as plsc

NUM_SIGNALS = 100000
ROW_WIDTH = 70
BATCH = 4096

_info = plsc.get_sparse_core_info()
_NC, _NS = _info.num_cores, _info.num_subcores
_NW = _NC * _NS
_UNROLL = 8
_NGRP = BATCH // (16 * _UNROLL)


def _make_gather():
    mesh = plsc.VectorSubcoreMesh(core_axis_name="c", subcore_axis_name="s")

    @functools.partial(
        pl.kernel,
        mesh=mesh,
        out_type=jax.ShapeDtypeStruct((ROW_WIDTH, BATCH), jnp.float32),
        scratch_types=[
            pltpu.VMEM((BATCH,), jnp.int32),
            pltpu.VMEM((NUM_SIGNALS,), jnp.float32),
            pltpu.VMEM((BATCH,), jnp.float32),
            pltpu.SemaphoreType.DMA,
        ],
        compiler_params=pltpu.CompilerParams(
            use_tc_tiling_on_sc=True,
            needs_layout_passes=False,
            skip_device_barrier=True,
        ),
    )
    def gather_kernel(idx_hbm, table_hbm, out_hbm, idx_v, plane_v, res_v, sem):
        wid = lax.axis_index("s") * _NC + lax.axis_index("c")
        pltpu.make_async_copy(table_hbm.at[wid], plane_v, sem).start()
        pltpu.sync_copy(idx_hbm, idx_v)

        def do_plane(j, fetch=True):
            if fetch:
                pltpu.make_async_copy(table_hbm.at[j], plane_v, sem).start()
            pltpu.make_async_copy(table_hbm.at[0], plane_v, sem).wait()

            @plsc.parallel_loop(0, BATCH, step=16, unroll=_UNROLL)
            def _(o):
                g = plsc.load_gather(plane_v, [idx_v[pl.ds(o, 16)]])
                res_v[pl.ds(o, 16)] = g

            pltpu.sync_copy(res_v, out_hbm.at[j])

        do_plane(wid, fetch=False)
        do_plane(wid + _NW)

        @pl.when(wid < ROW_WIDTH - 2 * _NW)
        def _():
            do_plane(wid + 2 * _NW)

    return gather_kernel


_gather = _make_gather()


def kernel(idx, appearance):
    out_t = _gather(idx.astype(jnp.int32), appearance.T)
    return out_t.T
